# Initial kernel scaffold; baseline (speedup 1.0000x reference)
import jax
import jax.numpy as jnp
from jax.experimental import pallas as pl


def kernel(locations, box_cls, box_regression, iou_scores):
    raise NotImplementedError("write your pallas kernel here")


# speedup vs baseline: 1.0075x; 1.0075x over previous
import jax
import jax.numpy as jnp
from jax.experimental import pallas as pl

PRE_NMS_THRESH = 0.3
PRE_NMS_TOP_N = 1000
DOWNSAMPLE = 32.0


def _mask_body(cls_ref, iou_ref, out_ref):
    c = cls_ref[...]
    u = iou_ref[...]
    s = 1.0 / ((1.0 + jnp.exp(-c)) * (1.0 + jnp.exp(-u)))
    out_ref[...] = jnp.where(s > PRE_NMS_THRESH, s, 0.0)


def kernel(locations, box_cls, box_regression, iou_scores):
    N, C, T = box_cls.shape
    BN = 8
    m = pl.pallas_call(
        _mask_body,
        grid=(N // BN,),
        in_specs=[
            pl.BlockSpec((BN, C, T), lambda g: (g, 0, 0)),
            pl.BlockSpec((BN, C, T), lambda g: (g, 0, 0)),
        ],
        out_specs=pl.BlockSpec((BN, C, T), lambda g: (g, 0, 0)),
        out_shape=jax.ShapeDtypeStruct((N, C, T), jnp.float32),
    )(box_cls, iou_scores)

    flat = jnp.transpose(m, (0, 2, 1)).reshape(N, T * C)
    topv, topi = jax.lax.top_k(flat, PRE_NMS_TOP_N)

    box_loc = topi // C
    labels = topi % C + 1

    reg = jnp.transpose(box_regression, (0, 2, 1))
    per_reg = jnp.take_along_axis(reg, box_loc[..., None], axis=1)
    per_loc = locations[box_loc]

    start = jnp.clip((per_loc - per_reg[..., 0]) / DOWNSAMPLE, 0.0, 1.0)
    end = jnp.clip((per_loc + per_reg[..., 1]) / DOWNSAMPLE, 0.0, 1.0)
    duration = end - start

    valid = (topv > PRE_NMS_THRESH) & (duration >= 0.0)
    vf = valid.astype(jnp.float32)

    safe = jnp.where(valid, topv, 1.0)
    scores = jnp.sqrt(safe) * vf
    detections = jnp.stack([start, end], axis=-1) * vf[..., None]
    norm_loc = (per_loc / DOWNSAMPLE) * vf
    return detections, scores, norm_loc, labels
